# Initial kernel scaffold; baseline (speedup 1.0000x reference)
import jax
import jax.numpy as jnp
from jax.experimental import pallas as pl


def kernel(loc_data, conf_data, target_loc, target_conf):
    raise NotImplementedError("write your pallas kernel here")


# speedup vs baseline: 1.2213x; 1.2213x over previous
import jax
import jax.numpy as jnp
from jax.experimental import pallas as pl

_B, _N, _C = 32, 20000, 81
_TN = 2000
_NB = _N // _TN


def _pass1_kernel(conf_ref, tcls_ref, loc_ref, tloc_ref, ce_ref, lloc_ref):
    b = pl.program_id(0)
    nb = pl.program_id(1)
    conf = conf_ref[0]
    tc = tcls_ref[0]
    m = jnp.max(conf, axis=1, keepdims=True)
    e = jnp.exp(conf - m)
    s = jnp.sum(e, axis=1, keepdims=True)
    lse = jnp.log(s) + m
    cls = jax.lax.broadcasted_iota(jnp.int32, (_TN, _C), 1)
    gath = jnp.sum(jnp.where(cls == tc, conf, 0.0), axis=1, keepdims=True)
    ce_ref[0] = lse - gath

    pos = (tc > 0).astype(jnp.float32)
    d = loc_ref[0] - tloc_ref[0]
    ad = jnp.abs(d)
    sl1 = jnp.where(ad < 1.0, 0.5 * d * d, ad - 0.5)
    part = jnp.sum(jnp.sum(sl1, axis=1, keepdims=True) * pos)
    part = part.reshape(1, 1)

    first = jnp.logical_and(b == 0, nb == 0)

    @pl.when(first)
    def _():
        lloc_ref[0] = part

    @pl.when(jnp.logical_not(first))
    def _():
        lloc_ref[0] += part


def _pass2_kernel(ce_ref, tcls_ref, out_ref):
    ce = ce_ref[...]
    tc = tcls_ref[...]
    pos = tc > 0
    posf = pos.astype(jnp.float32)
    num_pos = jnp.sum(posf, axis=1, keepdims=True)
    k = jnp.minimum(3.0 * num_pos, float(_N - 1))
    loss_c = jnp.where(pos, 0.0, ce)
    bits = jax.lax.bitcast_convert_type(loss_c, jnp.int32)

    def body(i, cand):
        trial = cand | (jnp.int32(1) << (30 - i))
        cnt = jnp.sum((bits >= trial).astype(jnp.float32), axis=1,
                      keepdims=True)
        return jnp.where(cnt >= k, trial, cand)

    cand = jax.lax.fori_loop(0, 31, body, jnp.zeros((_B, 1), jnp.int32))
    t = jax.lax.bitcast_convert_type(cand, jnp.float32)

    gt = loss_c > t
    cnt_gt = jnp.sum(gt.astype(jnp.float32), axis=1, keepdims=True)
    sum_gt = jnp.sum(jnp.where(gt, loss_c, 0.0), axis=1, keepdims=True)
    neg_c = jnp.where(k > 0, sum_gt + t * (k - cnt_gt), 0.0)
    pos_c = jnp.sum(jnp.where(pos, ce, 0.0), axis=1, keepdims=True)
    conf_sum = jnp.sum(pos_c + neg_c, axis=0, keepdims=True)
    ntot = jnp.sum(num_pos, axis=0, keepdims=True)
    out_ref[...] = jnp.concatenate([conf_sum, ntot], axis=1)


def kernel(loc_data, conf_data, target_loc, target_conf):
    b, n, c = conf_data.shape
    tc = target_conf.astype(jnp.int32)
    tc3 = tc.reshape(b, n, 1)

    ce3, lloc = pl.pallas_call(
        _pass1_kernel,
        grid=(b, n // _TN),
        in_specs=[
            pl.BlockSpec((1, _TN, c), lambda i, j: (i, j, 0)),
            pl.BlockSpec((1, _TN, 1), lambda i, j: (i, j, 0)),
            pl.BlockSpec((1, _TN, 4), lambda i, j: (i, j, 0)),
            pl.BlockSpec((1, _TN, 4), lambda i, j: (i, j, 0)),
        ],
        out_specs=[
            pl.BlockSpec((1, _TN, 1), lambda i, j: (i, j, 0)),
            pl.BlockSpec((1, 1, 1), lambda i, j: (0, 0, 0)),
        ],
        out_shape=[
            jax.ShapeDtypeStruct((b, n, 1), jnp.float32),
            jax.ShapeDtypeStruct((1, 1, 1), jnp.float32),
        ],
    )(conf_data, tc3, loc_data, target_loc)

    out = pl.pallas_call(
        _pass2_kernel,
        in_specs=[
            pl.BlockSpec((b, n), lambda: (0, 0)),
            pl.BlockSpec((b, n), lambda: (0, 0)),
        ],
        out_specs=pl.BlockSpec((1, 2), lambda: (0, 0)),
        out_shape=jax.ShapeDtypeStruct((1, 2), jnp.float32),
    )(ce3.reshape(b, n), tc)

    loss_loc = lloc[0, 0, 0]
    conf_sum = out[0, 0]
    n_tot = out[0, 1]
    return (loss_loc / n_tot, conf_sum / n_tot)
